# Initial kernel scaffold; baseline (speedup 1.0000x reference)
import functools

import jax
import jax.numpy as jnp
from jax import lax
from jax.experimental import pallas as pl
from jax.experimental.pallas import tpu as pltpu
from jax.experimental.pallas import tpu_sc as plsc

HALF = 128
CHUNK = 128
N_TILES = 16
N_CORES = 2


def _sc_aggregate(gsrc, dstp, feat2, n_nodes, e_pad):
    edges_per_tile = e_pad // N_TILES
    n_chunks = edges_per_tile // CHUNK
    acc_rows = ((n_nodes + 1 + N_TILES * CHUNK - 1)
                // (N_TILES * CHUNK)) * (N_TILES * CHUNK)
    zero_blocks = acc_rows // (N_TILES * CHUNK)
    out_rows_per_tile = n_nodes // N_TILES

    mesh = plsc.VectorSubcoreMesh(core_axis_name="c", subcore_axis_name="s")

    @functools.partial(
        pl.kernel,
        out_type=jax.ShapeDtypeStruct((N_CORES, n_nodes, HALF), jnp.float32),
        mesh=mesh,
        scratch_types=[
            pltpu.VMEM_SHARED((acc_rows, HALF), jnp.float32),
            pltpu.VMEM((2, CHUNK), jnp.int32),
            pltpu.VMEM((2, CHUNK), jnp.int32),
            pltpu.VMEM((2, CHUNK, HALF), jnp.float32),
            pltpu.SemaphoreType.DMA,
            pltpu.SemaphoreType.DMA,
        ],
    )
    def body(gsrc_hbm, dst_hbm, feat_hbm, out_hbm,
             acc, sidx, didx, rows, sem0, sem1):
        c = lax.axis_index("c")
        s = lax.axis_index("s")

        zvec = jnp.zeros((16,), jnp.float32)

        def zrow(i, carry):
            for j in range(HALF // 16):
                rows[0, i, pl.ds(j * 16, 16)] = zvec
            return carry

        lax.fori_loop(0, CHUNK, zrow, 0)
        zbase = s * (zero_blocks * CHUNK)
        for k in range(zero_blocks):
            pltpu.sync_copy(rows.at[0],
                            acc.at[pl.ds(zbase + k * CHUNK, CHUNK)])
        plsc.subcore_barrier()

        ebase = s * edges_per_tile

        def chunk_body(g, carry):
            off = ebase + g * CHUNK
            pltpu.sync_copy(gsrc_hbm.at[c, pl.ds(off, CHUNK)], sidx.at[0])
            pltpu.sync_copy(dst_hbm.at[pl.ds(off, CHUNK)], didx.at[0])
            pltpu.async_copy(feat_hbm.at[sidx.at[0]], rows.at[0], sem0).wait()
            pltpu.sync_copy(rows.at[0], acc.at[didx.at[0]], add=True)
            return carry

        lax.fori_loop(0, n_chunks, chunk_body, 0)
        plsc.subcore_barrier()

        ob = s * out_rows_per_tile
        pltpu.sync_copy(acc.at[pl.ds(ob, out_rows_per_tile)],
                        out_hbm.at[c, pl.ds(ob, out_rows_per_tile)])

    return body(gsrc, dstp, feat2)


def _tc_finish(agg, W, b):
    _, n, _ = agg.shape
    d_out = W.shape[1]
    block_rows = 1000

    def body(x_ref, w_ref, b_ref, o_ref):
        y = jnp.dot(x_ref[0], w_ref[:HALF, :],
                    preferred_element_type=jnp.float32)
        y = y + jnp.dot(x_ref[1], w_ref[HALF:, :],
                        preferred_element_type=jnp.float32)
        bb = b_ref[...]
        act = jnp.maximum(y + bb, 0.0)
        o_ref[...] = 2.0 * (act - bb)

    return pl.pallas_call(
        body,
        grid=(n // block_rows,),
        in_specs=[
            pl.BlockSpec((N_CORES, block_rows, HALF), lambda i: (0, i, 0)),
            pl.BlockSpec(W.shape, lambda i: (0, 0)),
            pl.BlockSpec(b.shape, lambda i: (0, 0)),
        ],
        out_specs=pl.BlockSpec((block_rows, d_out), lambda i: (i, 0)),
        out_shape=jax.ShapeDtypeStruct((n, d_out), jnp.float32),
    )(agg, W, b)


def kernel(features, edge_index, W, b):
    n, d = features.shape
    e = edge_index.shape[1]
    assert d == 2 * HALF

    src = edge_index[0].astype(jnp.int32)
    dst = edge_index[1].astype(jnp.int32)

    tile_quantum = N_TILES * CHUNK
    e_pad = ((e + tile_quantum - 1) // tile_quantum) * tile_quantum
    pad = e_pad - e
    src_p = jnp.concatenate([src, jnp.zeros((pad,), jnp.int32)])
    dst_p = jnp.concatenate([dst, jnp.full((pad,), n, jnp.int32)])
    gsrc = jnp.stack([2 * src_p, 2 * src_p + 1])

    feat2 = features.reshape(2 * n, HALF)
    agg = _sc_aggregate(gsrc, dst_p, feat2, n, e_pad)
    return _tc_finish(agg, W, b)


# speedup vs baseline: 3.5720x; 3.5720x over previous
import functools

import jax
import jax.numpy as jnp
from jax import lax
from jax.experimental import pallas as pl
from jax.experimental.pallas import tpu as pltpu
from jax.experimental.pallas import tpu_sc as plsc

HALF = 128
CHUNK = 128
N_TILES = 16
N_CORES = 2


def _sc_aggregate(gsrc, dstp, feat2, n_nodes, e_pad):
    edges_per_tile = e_pad // N_TILES
    n_chunks = edges_per_tile // CHUNK
    acc_rows = ((n_nodes + 1 + N_TILES * CHUNK - 1)
                // (N_TILES * CHUNK)) * (N_TILES * CHUNK)
    zero_blocks = acc_rows // (N_TILES * CHUNK)
    out_block = 400
    assert n_nodes % out_block == 0
    n_out_blocks = n_nodes // out_block
    out_rounds = (n_out_blocks + N_TILES - 1) // N_TILES

    mesh = plsc.VectorSubcoreMesh(core_axis_name="c", subcore_axis_name="s")

    @functools.partial(
        pl.kernel,
        out_type=jax.ShapeDtypeStruct((N_CORES, n_nodes, HALF), jnp.float32),
        mesh=mesh,
        scratch_types=[
            pltpu.VMEM_SHARED((acc_rows, HALF), jnp.float32),
            pltpu.VMEM((2, CHUNK), jnp.int32),
            pltpu.VMEM((2, CHUNK), jnp.int32),
            pltpu.VMEM((2, CHUNK, HALF), jnp.float32),
            pltpu.SemaphoreType.DMA,
            pltpu.SemaphoreType.DMA,
        ],
    )
    def body(gsrc_hbm, dst_hbm, feat_hbm, out_hbm,
             acc, sidx, didx, rows, sem0, sem1):
        c = lax.axis_index("c")
        s = lax.axis_index("s")

        zvec = jnp.zeros((16,), jnp.float32)

        def zrow(i, carry):
            for j in range(HALF // 16):
                rows[0, i, pl.ds(j * 16, 16)] = zvec
            return carry

        lax.fori_loop(0, CHUNK, zrow, 0)
        zbase = s * (zero_blocks * CHUNK)
        for k in range(zero_blocks):
            pltpu.sync_copy(rows.at[0],
                            acc.at[pl.ds(zbase + k * CHUNK, CHUNK)])
        plsc.subcore_barrier()

        ebase = s * edges_per_tile

        def chunk_body(g, carry):
            off = ebase + g * CHUNK
            pltpu.sync_copy(gsrc_hbm.at[c, pl.ds(off, CHUNK)], sidx.at[0])
            pltpu.sync_copy(dst_hbm.at[pl.ds(off, CHUNK)], didx.at[0])
            pltpu.async_copy(feat_hbm.at[sidx.at[0]], rows.at[0], sem0).wait()
            pltpu.sync_copy(rows.at[0], acc.at[didx.at[0]], add=True)
            return carry

        lax.fori_loop(0, n_chunks, chunk_body, 0)
        plsc.subcore_barrier()

        for k in range(out_rounds):
            blk = s + N_TILES * k

            @pl.when(blk < n_out_blocks)
            def _():
                pltpu.sync_copy(
                    acc.at[pl.ds(blk * out_block, out_block)],
                    out_hbm.at[c, pl.ds(blk * out_block, out_block)])

    return body(gsrc, dstp, feat2)


def _tc_finish(agg, W, b):
    _, n, _ = agg.shape
    d_out = W.shape[1]
    block_rows = 1000

    def body(x_ref, w_ref, b_ref, o_ref):
        y = jnp.dot(x_ref[0], w_ref[:HALF, :],
                    preferred_element_type=jnp.float32)
        y = y + jnp.dot(x_ref[1], w_ref[HALF:, :],
                        preferred_element_type=jnp.float32)
        bb = b_ref[...]
        act = jnp.maximum(y + bb, 0.0)
        o_ref[...] = 2.0 * (act - bb)

    return pl.pallas_call(
        body,
        grid=(n // block_rows,),
        in_specs=[
            pl.BlockSpec((N_CORES, block_rows, HALF), lambda i: (0, i, 0)),
            pl.BlockSpec(W.shape, lambda i: (0, 0)),
            pl.BlockSpec(b.shape, lambda i: (0, 0)),
        ],
        out_specs=pl.BlockSpec((block_rows, d_out), lambda i: (i, 0)),
        out_shape=jax.ShapeDtypeStruct((n, d_out), jnp.float32),
    )(agg, W, b)


def kernel(features, edge_index, W, b):
    n, d = features.shape
    e = edge_index.shape[1]
    assert d == 2 * HALF

    src = edge_index[0].astype(jnp.int32)
    dst = edge_index[1].astype(jnp.int32)

    tile_quantum = N_TILES * CHUNK
    e_pad = ((e + tile_quantum - 1) // tile_quantum) * tile_quantum
    pad = e_pad - e
    src_p = jnp.concatenate([src, jnp.zeros((pad,), jnp.int32)])
    dst_p = jnp.concatenate([dst, jnp.full((pad,), n, jnp.int32)])
    gsrc = jnp.stack([2 * src_p, 2 * src_p + 1])

    feat2 = features.reshape(2 * n, HALF)
    agg = _sc_aggregate(gsrc, dst_p, feat2, n, e_pad)
    return _tc_finish(agg, W, b)


# speedup vs baseline: 3.8340x; 1.0733x over previous
import functools

import jax
import jax.numpy as jnp
from jax import lax
from jax.experimental import pallas as pl
from jax.experimental.pallas import tpu as pltpu
from jax.experimental.pallas import tpu_sc as plsc

HALF = 128
CHUNK = 128
N_TILES = 16
N_CORES = 2


def _sc_aggregate(gsrc, dstp, feat2, n_nodes):
    n_chunks_total = dstp.shape[0]
    n_chunks = n_chunks_total // N_TILES
    n_phases = 2
    cpp = n_chunks // n_phases
    acc_rows = ((n_nodes + 1 + N_TILES * CHUNK - 1)
                // (N_TILES * CHUNK)) * (N_TILES * CHUNK)
    zero_blocks = acc_rows // (N_TILES * CHUNK)
    out_block = 400
    assert n_nodes % out_block == 0
    n_out_blocks = n_nodes // out_block
    out_rounds = (n_out_blocks + N_TILES - 1) // N_TILES

    mesh = plsc.VectorSubcoreMesh(core_axis_name="c", subcore_axis_name="s")

    @functools.partial(
        pl.kernel,
        out_type=jax.ShapeDtypeStruct((N_CORES, n_nodes, HALF), jnp.float32),
        mesh=mesh,
        scratch_types=[
            pltpu.VMEM_SHARED((acc_rows, HALF), jnp.float32),
            pltpu.VMEM((cpp, CHUNK), jnp.int32),
            pltpu.VMEM((cpp, CHUNK), jnp.int32),
            pltpu.VMEM((2, CHUNK, HALF), jnp.float32),
            pltpu.SemaphoreType.DMA,
            pltpu.SemaphoreType.DMA,
        ],
    )
    def body(gsrc_hbm, dst_hbm, feat_hbm, out_hbm,
             acc, sidx, didx, rows, sem0, sem1):
        c = lax.axis_index("c")
        s = lax.axis_index("s")
        sems = (sem0, sem1)
        cbase = s * n_chunks

        zvec = jnp.zeros((16,), jnp.float32)

        def zrow(i, carry):
            for j in range(HALF // 16):
                rows[0, i, pl.ds(j * 16, 16)] = zvec
            return carry

        lax.fori_loop(0, CHUNK, zrow, 0)
        zbase = s * (zero_blocks * CHUNK)
        for k in range(zero_blocks):
            pltpu.sync_copy(rows.at[0],
                            acc.at[pl.ds(zbase + k * CHUNK, CHUNK)])
        plsc.subcore_barrier()

        def gather_start(g, buf):
            pltpu.async_copy(feat_hbm.at[sidx.at[g]], rows.at[buf],
                             sems[buf])

        def gather_wait(g, buf):
            pltpu.make_async_copy(feat_hbm.at[sidx.at[g]], rows.at[buf],
                                  sems[buf]).wait()

        def scatter_add(g, buf):
            pltpu.sync_copy(rows.at[buf], acc.at[didx.at[g]], add=True)

        for p in range(n_phases):
            pbase = cbase + p * cpp
            pltpu.sync_copy(gsrc_hbm.at[c, pl.ds(pbase, cpp)], sidx)
            pltpu.sync_copy(dst_hbm.at[pl.ds(pbase, cpp)], didx)

            gather_start(0, 0)

            def pair_body(i, carry):
                for h in range(2):
                    g = 2 * i + h
                    nxt = g + 1

                    @pl.when(nxt < cpp)
                    def _():
                        gather_start(nxt, 1 - h)

                    gather_wait(g, h)
                    scatter_add(g, h)
                return carry

            lax.fori_loop(0, cpp // 2, pair_body, 0)
        plsc.subcore_barrier()

        for k in range(out_rounds):
            blk = s + N_TILES * k

            @pl.when(blk < n_out_blocks)
            def _():
                pltpu.sync_copy(
                    acc.at[pl.ds(blk * out_block, out_block)],
                    out_hbm.at[c, pl.ds(blk * out_block, out_block)])

    return body(gsrc, dstp, feat2)


def _tc_finish(agg, W, b):
    _, n, _ = agg.shape
    d_out = W.shape[1]
    block_rows = 1000

    def body(x_ref, w_ref, b_ref, o_ref):
        y = jnp.dot(x_ref[0], w_ref[:HALF, :],
                    preferred_element_type=jnp.float32)
        y = y + jnp.dot(x_ref[1], w_ref[HALF:, :],
                        preferred_element_type=jnp.float32)
        bb = b_ref[...]
        act = jnp.maximum(y + bb, 0.0)
        o_ref[...] = 2.0 * (act - bb)

    return pl.pallas_call(
        body,
        grid=(n // block_rows,),
        in_specs=[
            pl.BlockSpec((N_CORES, block_rows, HALF), lambda i: (0, i, 0)),
            pl.BlockSpec(W.shape, lambda i: (0, 0)),
            pl.BlockSpec(b.shape, lambda i: (0, 0)),
        ],
        out_specs=pl.BlockSpec((block_rows, d_out), lambda i: (i, 0)),
        out_shape=jax.ShapeDtypeStruct((n, d_out), jnp.float32),
    )(agg, W, b)


def kernel(features, edge_index, W, b):
    n, d = features.shape
    e = edge_index.shape[1]
    assert d == 2 * HALF

    src = edge_index[0].astype(jnp.int32)
    dst = edge_index[1].astype(jnp.int32)

    tile_quantum = 4 * N_TILES * CHUNK
    e_pad = ((e + tile_quantum - 1) // tile_quantum) * tile_quantum
    pad = e_pad - e
    src_p = jnp.concatenate([src, jnp.zeros((pad,), jnp.int32)])
    dst_p = jnp.concatenate([dst, jnp.full((pad,), n, jnp.int32)])
    gsrc = jnp.stack([2 * src_p, 2 * src_p + 1]).reshape(2, -1, CHUNK)
    dst_p = dst_p.reshape(-1, CHUNK)

    feat2 = features.reshape(2 * n, HALF)
    agg = _sc_aggregate(gsrc, dst_p, feat2, n)
    return _tc_finish(agg, W, b)


# speedup vs baseline: 3.8364x; 1.0006x over previous
import functools

import jax
import jax.numpy as jnp
from jax import lax
from jax.experimental import pallas as pl
from jax.experimental.pallas import tpu as pltpu
from jax.experimental.pallas import tpu_sc as plsc

HALF = 128
CHUNK = 128
N_TILES = 16
N_CORES = 2


def _sc_aggregate(gsrc, dstp, feat2, n_nodes):
    n_chunks_total = dstp.shape[0]
    n_chunks = n_chunks_total // N_TILES
    n_phases = 2
    cpp = n_chunks // n_phases
    acc_rows = ((n_nodes + 1 + N_TILES * CHUNK - 1)
                // (N_TILES * CHUNK)) * (N_TILES * CHUNK)
    zero_blocks = acc_rows // (N_TILES * CHUNK)
    out_block = 400
    assert n_nodes % out_block == 0
    n_out_blocks = n_nodes // out_block
    out_rounds = (n_out_blocks + N_TILES - 1) // N_TILES

    mesh = plsc.VectorSubcoreMesh(core_axis_name="c", subcore_axis_name="s")

    @functools.partial(
        pl.kernel,
        out_type=jax.ShapeDtypeStruct((N_CORES, n_nodes, HALF), jnp.float32),
        mesh=mesh,
        scratch_types=[
            pltpu.VMEM_SHARED((acc_rows, HALF), jnp.float32),
            pltpu.VMEM((cpp, CHUNK), jnp.int32),
            pltpu.VMEM((cpp, CHUNK), jnp.int32),
            pltpu.VMEM((2, CHUNK, HALF), jnp.float32),
            pltpu.SemaphoreType.DMA,
            pltpu.SemaphoreType.DMA,
        ],
    )
    def body(gsrc_hbm, dst_hbm, feat_hbm, out_hbm,
             acc, sidx, didx, rows, sem0, sem1):
        c = lax.axis_index("c")
        s = lax.axis_index("s")
        sems = (sem0, sem1)
        cbase = s * n_chunks

        zvec = jnp.zeros((16,), jnp.float32)

        def zrow(i, carry):
            for j in range(HALF // 16):
                rows[0, i, pl.ds(j * 16, 16)] = zvec
            return carry

        lax.fori_loop(0, CHUNK, zrow, 0)
        zbase = s * (zero_blocks * CHUNK)
        for k in range(zero_blocks):
            pltpu.sync_copy(rows.at[0],
                            acc.at[pl.ds(zbase + k * CHUNK, CHUNK)])
        plsc.subcore_barrier()

        SUB = CHUNK // 2

        def gather_start(g, buf):
            pltpu.async_copy(feat_hbm.at[sidx.at[g, pl.ds(0, SUB)]],
                             rows.at[buf, pl.ds(0, SUB)], sems[buf])
            pltpu.async_copy(feat_hbm.at[sidx.at[g, pl.ds(SUB, SUB)]],
                             rows.at[buf, pl.ds(SUB, SUB)], sems[buf])

        def gather_wait(g, buf):
            pltpu.make_async_copy(feat_hbm.at[sidx.at[g, pl.ds(0, SUB)]],
                                  rows.at[buf, pl.ds(0, SUB)],
                                  sems[buf]).wait()
            pltpu.make_async_copy(feat_hbm.at[sidx.at[g, pl.ds(SUB, SUB)]],
                                  rows.at[buf, pl.ds(SUB, SUB)],
                                  sems[buf]).wait()

        def scatter_add(g, buf):
            pltpu.sync_copy(rows.at[buf], acc.at[didx.at[g]], add=True)

        for p in range(n_phases):
            pbase = cbase + p * cpp
            pltpu.sync_copy(gsrc_hbm.at[c, pl.ds(pbase, cpp)], sidx)
            pltpu.sync_copy(dst_hbm.at[pl.ds(pbase, cpp)], didx)

            gather_start(0, 0)

            def pair_body(i, carry):
                for h in range(2):
                    g = 2 * i + h
                    nxt = g + 1

                    @pl.when(nxt < cpp)
                    def _():
                        gather_start(nxt, 1 - h)

                    gather_wait(g, h)
                    scatter_add(g, h)
                return carry

            lax.fori_loop(0, cpp // 2, pair_body, 0)
        plsc.subcore_barrier()

        for k in range(out_rounds):
            blk = s + N_TILES * k

            @pl.when(blk < n_out_blocks)
            def _():
                pltpu.sync_copy(
                    acc.at[pl.ds(blk * out_block, out_block)],
                    out_hbm.at[c, pl.ds(blk * out_block, out_block)])

    return body(gsrc, dstp, feat2)


def _tc_finish(agg, W, b):
    _, n, _ = agg.shape
    d_out = W.shape[1]
    block_rows = 1000

    def body(x_ref, w_ref, b_ref, o_ref):
        y = jnp.dot(x_ref[0], w_ref[:HALF, :],
                    preferred_element_type=jnp.float32)
        y = y + jnp.dot(x_ref[1], w_ref[HALF:, :],
                        preferred_element_type=jnp.float32)
        bb = b_ref[...]
        act = jnp.maximum(y + bb, 0.0)
        o_ref[...] = 2.0 * (act - bb)

    return pl.pallas_call(
        body,
        grid=(n // block_rows,),
        in_specs=[
            pl.BlockSpec((N_CORES, block_rows, HALF), lambda i: (0, i, 0)),
            pl.BlockSpec(W.shape, lambda i: (0, 0)),
            pl.BlockSpec(b.shape, lambda i: (0, 0)),
        ],
        out_specs=pl.BlockSpec((block_rows, d_out), lambda i: (i, 0)),
        out_shape=jax.ShapeDtypeStruct((n, d_out), jnp.float32),
    )(agg, W, b)


def kernel(features, edge_index, W, b):
    n, d = features.shape
    e = edge_index.shape[1]
    assert d == 2 * HALF

    src = edge_index[0].astype(jnp.int32)
    dst = edge_index[1].astype(jnp.int32)

    tile_quantum = 4 * N_TILES * CHUNK
    e_pad = ((e + tile_quantum - 1) // tile_quantum) * tile_quantum
    pad = e_pad - e
    src_p = jnp.concatenate([src, jnp.zeros((pad,), jnp.int32)])
    dst_p = jnp.concatenate([dst, jnp.full((pad,), n, jnp.int32)])
    gsrc = jnp.stack([2 * src_p, 2 * src_p + 1]).reshape(2, -1, CHUNK)
    dst_p = dst_p.reshape(-1, CHUNK)

    feat2 = features.reshape(2 * n, HALF)
    agg = _sc_aggregate(gsrc, dst_p, feat2, n)
    return _tc_finish(agg, W, b)
